# Initial kernel scaffold; baseline (speedup 1.0000x reference)
import jax
import jax.numpy as jnp
from jax.experimental import pallas as pl


def kernel(x, w1, b1, w2, b2, w3, b3, out1, in1, out2, in2, out3, in3):
    raise NotImplementedError("write your pallas kernel here")


# speedup vs baseline: 3.2358x; 3.2358x over previous
import functools

import jax
import jax.numpy as jnp
from jax import lax
from jax.experimental import pallas as pl
from jax.experimental.pallas import tpu as pltpu
from jax.experimental.pallas import tpu_sc as plsc

N_NODES = 10000
N_EDGES = 160000
N_TRI = 100000
N_TET = 20000
B = 32

NUM_CORES = 2
NUM_SUBCORES = 16
NW = NUM_CORES * NUM_SUBCORES
CHUNK = 256
IR = 128


def _make_layer(fan: int, n_pad: int):
  m = n_pad // (CHUNK * NW)
  FK = fan * CHUNK
  R = FK // IR
  assert m >= 3
  mesh = plsc.VectorSubcoreMesh(core_axis_name="core", subcore_axis_name="subcore",
                                num_cores=NUM_CORES, num_subcores=NUM_SUBCORES)

  def body(table, idxm, w, b, out, idx_v, rows_v, w_v, b_v, out_v,
           sem_in, sem_g, sem_out):
    wid = lax.axis_index("subcore") * NUM_CORES + lax.axis_index("core")
    iota = lax.iota(jnp.int32, 16)
    base_chunk = wid * m

    def in_copies(t, slot, fire):
      chunk = base_chunk + t
      pairs = [
          (idxm.at[pl.ds(chunk * R, R)], idx_v.at[pl.ds(slot * R, R)]),
          (w.at[pl.ds(chunk * FK, FK)], w_v.at[pl.ds(slot * FK, FK)]),
          (b.at[pl.ds(chunk * CHUNK, CHUNK)], b_v.at[pl.ds(slot * CHUNK, CHUNK)]),
      ]
      for s_, d_ in pairs:
        cp = pltpu.make_async_copy(s_, d_, sem_in.at[slot])
        if fire:
          cp.start()
        else:
          cp.wait()

    def gathers(t, slot, fire):
      for j in range(R):
        cp = pltpu.make_async_copy(
            table.at[idx_v.at[slot * R + j]],
            rows_v.at[pl.ds(slot * FK + j * IR, IR)],
            sem_g.at[slot])
        if fire:
          cp.start()
        else:
          cp.wait()

    def out_copy(t, slot, fire):
      chunk = base_chunk + t
      cp = pltpu.make_async_copy(out_v.at[pl.ds(slot * CHUNK, CHUNK)],
                                 out.at[pl.ds(chunk * CHUNK, CHUNK)],
                                 sem_out.at[slot])
      if fire:
        cp.start()
      else:
        cp.wait()

    def compute(slot):
      def group(g, carry):
        gb = g * 16
        oid = gb + iota
        base = slot * FK + oid * fan
        bias = b_v[pl.ds(slot * CHUNK + gb, 16)]
        wjs = [plsc.load_gather(w_v, [base + j]) for j in range(fan)]
        out_idx = slot * CHUNK + oid
        for c in range(B):
          cvec = jnp.full((16,), c, jnp.int32)
          acc = bias
          for j in range(fan):
            acc = acc + wjs[j] * plsc.load_gather(rows_v, [base + j, cvec])
          acc = jnp.maximum(acc, 0.0)
          plsc.store_scatter(out_v, [out_idx, cvec], acc)
        return carry
      lax.fori_loop(0, CHUNK // 16, group, 0)

    in_copies(0, 0, True)
    in_copies(1, 1, True)
    in_copies(0, 0, False)
    gathers(0, 0, True)

    def body_t(t, carry):
      slot = lax.rem(t, 2)
      nslot = 1 - slot
      gathers(t, slot, False)

      @pl.when(t + 1 < m)
      def _():
        in_copies(t + 1, nslot, False)
        gathers(t + 1, nslot, True)

      @pl.when(t >= 2)
      def _():
        out_copy(t - 2, slot, False)

      compute(slot)
      out_copy(t, slot, True)

      @pl.when(t + 2 < m)
      def _():
        in_copies(t + 2, slot, True)
      return carry

    lax.fori_loop(0, m, body_t, 0)
    out_copy(m - 2, (m - 2) % 2, False)
    out_copy(m - 1, (m - 1) % 2, False)

  return pl.kernel(
      body,
      out_type=jax.ShapeDtypeStruct((n_pad, B), jnp.float32),
      mesh=mesh,
      compiler_params=pltpu.CompilerParams(needs_layout_passes=False,
                                           use_tc_tiling_on_sc=False),
      scratch_types=[
          pltpu.VMEM((2 * R, IR), jnp.int32),
          pltpu.VMEM((2 * FK, B), jnp.float32),
          pltpu.VMEM((2 * FK,), jnp.float32),
          pltpu.VMEM((2 * CHUNK,), jnp.float32),
          pltpu.VMEM((2 * CHUNK, B), jnp.float32),
          pltpu.SemaphoreType.DMA((2,)),
          pltpu.SemaphoreType.DMA((2,)),
          pltpu.SemaphoreType.DMA((2,)),
      ],
  )


def _pad_up(n: int) -> int:
  q = CHUNK * NW
  return ((n + q - 1) // q) * q


_NP1, _NP2, _NP3 = _pad_up(N_EDGES), _pad_up(N_TRI), _pad_up(N_TET)


@functools.lru_cache(maxsize=None)
def _layers():
  return _make_layer(2, _NP1), _make_layer(3, _NP2), _make_layer(4, _NP3)


def _prep(idx, w, b, fan, n_out, n_pad):
  idx_p = jnp.pad(idx, (0, fan * (n_pad - n_out))).reshape(n_pad * fan // IR, IR)
  w_p = jnp.pad(w, (0, fan * (n_pad - n_out)))
  b_p = jnp.pad(b, (0, n_pad - n_out))
  return idx_p, w_p, b_p


def kernel(x, w1, b1, w2, b2, w3, b3, out1, in1, out2, in2, out3, in3):
  del out1, out2, out3
  xt = x.T
  i1, wp1, bp1 = _prep(in1, w1, b1, 2, N_EDGES, _NP1)
  i2, wp2, bp2 = _prep(in2, w2, b2, 3, N_TRI, _NP2)
  i3, wp3, bp3 = _prep(in3, w3, b3, 4, N_TET, _NP3)
  l1, l2, l3 = _layers()
  y1 = l1(xt, i1, wp1, bp1)
  y2 = l2(y1, i2, wp2, bp2)
  y3 = l3(y2, i3, wp3, bp3)
  return jnp.concatenate([y1[:N_EDGES], y2[:N_TRI], y3[:N_TET]], axis=0).T
